# baseline (speedup 1.0000x reference)
import functools

import jax
import jax.numpy as jnp
from jax import lax
from jax.experimental import pallas as pl
from jax.experimental.pallas import tpu as pltpu
from jax.experimental.pallas import tpu_sc as plsc

B = 16384
D = 64
W = 2 * D
NC = 2
NS = 16
L = 16
NW = NC * NS
BPW = B // NW
C = 128
NCH = BPW // C

_mesh = plsc.VectorSubcoreMesh(core_axis_name="c", subcore_axis_name="s")
_params = pltpu.CompilerParams(
    needs_layout_passes=False, use_tc_tiling_on_sc=True
)


@functools.partial(
    pl.kernel,
    mesh=_mesh,
    compiler_params=_params,
    out_type=jax.ShapeDtypeStruct((B, D), jnp.float32),
    scratch_types=[
        pltpu.VMEM((BPW,), jnp.int32),
        pltpu.VMEM((C, W), jnp.float32),
        pltpu.VMEM((C, D), jnp.float32),
        pltpu.SemaphoreType.DMA,
    ],
)
def _gather_u(user_hbm, eu2_hbm, gu_hbm, sid, buf, sel, sem):
    wid = lax.axis_index("s") * NC + lax.axis_index("c")
    base = wid * BPW

    pltpu.sync_copy(user_hbm.at[pl.ds(base, BPW)], sid)

    def chunk(c, carry):
        cb = c * C

        def fire(s, carry2):
            sb = s * L
            vu = sid[pl.ds(cb + sb, L)]
            for k in range(L):
                pltpu.make_async_copy(
                    eu2_hbm.at[pl.ds(lax.shift_right_logical(vu[k], 1), 1), :],
                    buf.at[pl.ds(sb + k, 1), :], sem
                ).start()
            return carry2

        lax.fori_loop(0, C // L, fire, 0)
        pltpu.make_async_copy(eu2_hbm.at[pl.ds(0, C), :], buf, sem).wait()

        def pick(s, carry2):
            sb = s * L
            vu = sid[pl.ds(cb + sb, L)]
            for k in range(L):
                off = lax.bitwise_and(vu[k], 1) * D
                for q in range(D // L):
                    sel[sb + k, pl.ds(q * L, L)] = (
                        buf[sb + k, pl.ds(off + q * L, L)]
                    )
            return carry2

        lax.fori_loop(0, C // L, pick, 0)
        pltpu.sync_copy(sel, gu_hbm.at[pl.ds(base + cb, C), :])
        return carry

    lax.fori_loop(0, NCH, chunk, 0)


@functools.partial(
    pl.kernel,
    mesh=_mesh,
    compiler_params=_params,
    out_type=[
        jax.ShapeDtypeStruct((B,), jnp.float32),
        jax.ShapeDtypeStruct((B,), jnp.float32),
    ],
    scratch_types=[
        pltpu.VMEM((BPW,), jnp.int32),
        pltpu.VMEM((BPW,), jnp.int32),
        pltpu.VMEM((C, W), jnp.float32),
        pltpu.VMEM((C, W), jnp.float32),
        pltpu.VMEM((C, D), jnp.float32),
        pltpu.VMEM((BPW,), jnp.float32),
        pltpu.VMEM((BPW,), jnp.float32),
        pltpu.SemaphoreType.DMA,
    ],
)
def _gather_dot(item_i_hbm, item_j_hbm, ei2_hbm, gu_hbm,
                out_i_hbm, out_j_hbm,
                sid_i, sid_j, buf_i, buf_j, gub, pred_i, pred_j, sem):
    wid = lax.axis_index("s") * NC + lax.axis_index("c")
    base = wid * BPW

    pltpu.sync_copy(item_i_hbm.at[pl.ds(base, BPW)], sid_i)
    pltpu.sync_copy(item_j_hbm.at[pl.ds(base, BPW)], sid_j)

    lane = lax.iota(jnp.int32, L)

    def chunk(c, carry):
        cb = c * C

        def fire(s, carry2):
            sb = s * L
            vi = sid_i[pl.ds(cb + sb, L)]
            vj = sid_j[pl.ds(cb + sb, L)]
            for k in range(L):
                pltpu.make_async_copy(
                    ei2_hbm.at[pl.ds(lax.shift_right_logical(vi[k], 1), 1), :],
                    buf_i.at[pl.ds(sb + k, 1), :], sem
                ).start()
                pltpu.make_async_copy(
                    ei2_hbm.at[pl.ds(lax.shift_right_logical(vj[k], 1), 1), :],
                    buf_j.at[pl.ds(sb + k, 1), :], sem
                ).start()
            return carry2

        cg = pltpu.async_copy(gu_hbm.at[pl.ds(base + cb, C), :], gub, sem)
        lax.fori_loop(0, C // L, fire, 0)
        cg.wait()
        pltpu.make_async_copy(ei2_hbm.at[pl.ds(0, C), :], buf_i, sem).wait()
        pltpu.make_async_copy(ei2_hbm.at[pl.ds(0, C), :], buf_j, sem).wait()

        def group(g, carry2):
            rg = g * L
            vi = sid_i[pl.ds(cb + rg, L)]
            vj = sid_j[pl.ds(cb + rg, L)]
            out_i = jnp.zeros((L,), jnp.float32)
            out_j = jnp.zeros((L,), jnp.float32)
            for k in range(L):
                offi = lax.bitwise_and(vi[k], 1) * D
                offj = lax.bitwise_and(vj[k], 1) * D
                acc_i = jnp.zeros((L,), jnp.float32)
                acc_j = jnp.zeros((L,), jnp.float32)
                for q in range(D // L):
                    u = gub[rg + k, pl.ds(q * L, L)]
                    xi = buf_i[rg + k, pl.ds(offi + q * L, L)]
                    xj = buf_j[rg + k, pl.ds(offj + q * L, L)]
                    acc_i = acc_i + u * xi
                    acc_j = acc_j + u * xj
                out_i = jnp.where(lane == k, jnp.sum(acc_i), out_i)
                out_j = jnp.where(lane == k, jnp.sum(acc_j), out_j)
            pred_i[pl.ds(cb + rg, L)] = out_i
            pred_j[pl.ds(cb + rg, L)] = out_j
            return carry2

        lax.fori_loop(0, C // L, group, 0)
        return carry

    lax.fori_loop(0, NCH, chunk, 0)

    pltpu.sync_copy(pred_i, out_i_hbm.at[pl.ds(base, BPW)])
    pltpu.sync_copy(pred_j, out_j_hbm.at[pl.ds(base, BPW)])


def kernel(user, item_i, item_j, embed_user, embed_item):
    eu2 = embed_user.reshape(embed_user.shape[0] // 2, W)
    ei2 = embed_item.reshape(embed_item.shape[0] // 2, W)
    gu = _gather_u(user, eu2)
    out_i, out_j = _gather_dot(item_i, item_j, ei2, gu)
    return (out_i, out_j)
